# Initial kernel scaffold; baseline (speedup 1.0000x reference)
import jax
import jax.numpy as jnp
from jax.experimental import pallas as pl


def kernel(states, action, We1, be1, We2, be2, ge, bbe, We3, be3, Wn1, bn1, Wn2, bn2, gn, bbn, Wn3, bn3):
    raise NotImplementedError("write your pallas kernel here")


# speedup vs baseline: 18.5145x; 18.5145x over previous
import functools

import jax
import jax.numpy as jnp
from jax.experimental import pallas as pl

_A = 4


def _gnn_block(x_ref, act_ref, ws_ref, wt_ref, be1_ref, we2_ref, be2_ref,
               ge_ref, bbe_ref, we3_ref, be3_ref, wn1x_ref, wn1a_ref,
               wn1g_ref, bn1_ref, wn2_ref, bn2_ref, gn_ref, bbn_ref,
               wn3_ref, bn3_ref, out_ref, *, s, n, d, h):
    f32 = jnp.float32
    x = x_ref[...].reshape(s * n, d)

    u = jnp.dot(x, ws_ref[...], preferred_element_type=f32) + be1_ref[...]
    v = jnp.dot(x, wt_ref[...], preferred_element_type=f32)
    h1 = jnp.maximum(u.reshape(s, n, 1, h) + v.reshape(s, 1, n, h), 0.0)
    h1 = h1.reshape(s * n * n, h)

    e2 = jnp.dot(h1, we2_ref[...], preferred_element_type=f32) + be2_ref[...]
    mu = jnp.mean(e2, axis=-1, keepdims=True)
    var = jnp.mean((e2 - mu) ** 2, axis=-1, keepdims=True)
    e = (e2 - mu) * jax.lax.rsqrt(var + 1e-5) * ge_ref[...] + bbe_ref[...]
    e = jnp.maximum(e, 0.0)
    e3 = jnp.dot(e, we3_ref[...], preferred_element_type=f32) + be3_ref[...]

    e3 = e3.reshape(s * n, n, h)
    i_idx = jax.lax.broadcasted_iota(jnp.int32, (s * n, n), 0) % n
    j_idx = jax.lax.broadcasted_iota(jnp.int32, (s * n, n), 1)
    mask = (i_idx != j_idx).astype(f32)
    agg = jnp.sum(e3 * mask[:, :, None], axis=1)

    a = act_ref[0, 0, :]
    a_div = a // _A
    a_mod = a - a_div * _A
    mod_oh = (jax.lax.broadcasted_iota(jnp.int32, (s, _A), 1)
              == a_mod[:, None]).astype(f32)
    w_pick = jnp.dot(mod_oh, wn1a_ref[...], preferred_element_type=f32)
    node_oh = (jax.lax.broadcasted_iota(jnp.int32, (s, n), 1)
               == a_div[:, None]).astype(f32)
    act_add = (node_oh[:, :, None] * w_pick[:, None, :]).reshape(s * n, h)

    p = (jnp.dot(x, wn1x_ref[...], preferred_element_type=f32)
         + jnp.dot(agg, wn1g_ref[...], preferred_element_type=f32)
         + act_add + bn1_ref[...])
    hh = jnp.maximum(p, 0.0)
    h2 = jnp.dot(hh, wn2_ref[...], preferred_element_type=f32) + bn2_ref[...]
    mu2 = jnp.mean(h2, axis=-1, keepdims=True)
    var2 = jnp.mean((h2 - mu2) ** 2, axis=-1, keepdims=True)
    hn = (h2 - mu2) * jax.lax.rsqrt(var2 + 1e-5) * gn_ref[...] + bbn_ref[...]
    hn = jnp.maximum(hn, 0.0)
    out = jnp.dot(hn, wn3_ref[...], preferred_element_type=f32) + bn3_ref[...]
    out_ref[...] = out.reshape(s, n, out.shape[-1])


def kernel(states, action, We1, be1, We2, be2, ge, bbe, We3, be3,
           Wn1, bn1, Wn2, bn2, gn, bbn, Wn3, bn3):
    b, n, d = states.shape
    h = We2.shape[0]
    s = 32
    nb = b // s

    ws, wt = We1[:d], We1[d:]
    wn1x = Wn1[:d]
    wn1a = Wn1[d:d + _A]
    wn1g = Wn1[d + _A:]
    act3 = action.astype(jnp.int32).reshape(nb, 1, s)

    row = lambda z: z.reshape(1, -1)
    full = lambda shp: pl.BlockSpec(shp, lambda i: (0,) * len(shp))

    out = pl.pallas_call(
        functools.partial(_gnn_block, s=s, n=n, d=d, h=h),
        grid=(nb,),
        in_specs=[
            pl.BlockSpec((s, n, d), lambda i: (i, 0, 0)),
            pl.BlockSpec((1, 1, s), lambda i: (i, 0, 0)),
            full((d, h)), full((d, h)), full((1, h)),
            full((h, h)), full((1, h)),
            full((1, h)), full((1, h)),
            full((h, h)), full((1, h)),
            full((d, h)), full((_A, h)), full((h, h)),
            full((1, h)),
            full((h, h)), full((1, h)),
            full((1, h)), full((1, h)),
            full((h, d)), full((1, d)),
        ],
        out_specs=pl.BlockSpec((s, n, d), lambda i: (i, 0, 0)),
        out_shape=jax.ShapeDtypeStruct((b, n, d), jnp.float32),
    )(states, act3, ws, wt, row(be1), We2, row(be2), row(ge), row(bbe),
      We3, row(be3), wn1x, wn1a, wn1g, row(bn1), Wn2, row(bn2), row(gn),
      row(bbn), Wn3, row(bn3))
    return out


# speedup vs baseline: 19.0029x; 1.0264x over previous
import functools

import jax
import jax.numpy as jnp
from jax.experimental import pallas as pl

_A = 4


def _gnn_block(x_ref, act_ref, ws_ref, wt_ref, be1_ref, we2_ref, be2_ref,
               ge_ref, bbe_ref, we3_ref, be3_ref, wn1x_ref, wn1a_ref,
               wn1g_ref, bn1_ref, wn2_ref, bn2_ref, gn_ref, bbn_ref,
               wn3_ref, bn3_ref, out_ref, *, s, n, d, h):
    f32 = jnp.float32
    x = x_ref[...].reshape(s * n, d)

    u = jnp.dot(x, ws_ref[...], preferred_element_type=f32) + be1_ref[...]
    v = jnp.dot(x, wt_ref[...], preferred_element_type=f32)
    h1 = jnp.maximum(u.reshape(s, n, 1, h) + v.reshape(s, 1, n, h), 0.0)
    h1 = h1.reshape(s * n * n, h)

    def ln_relu(z, g, bb):
        mu = jnp.mean(z, axis=-1, keepdims=True)
        var = jnp.mean((z - mu) ** 2, axis=-1, keepdims=True)
        return jnp.maximum((z - mu) * jax.lax.rsqrt(var + 1e-5) * g + bb, 0.0)

    e2 = jnp.dot(h1, we2_ref[...], preferred_element_type=f32) + be2_ref[...]
    e = ln_relu(e2, ge_ref[...], bbe_ref[...])

    esum = jnp.sum(e.reshape(s * n, n, h), axis=1)
    d1 = jnp.maximum(u + v, 0.0)
    d2 = jnp.dot(d1, we2_ref[...], preferred_element_type=f32) + be2_ref[...]
    ed = ln_relu(d2, ge_ref[...], bbe_ref[...])
    agg = (jnp.dot(esum - ed, we3_ref[...], preferred_element_type=f32)
           + (n - 1) * be3_ref[...])

    a = act_ref[0, 0, :]
    a_div = a // _A
    a_mod = a - a_div * _A
    mod_oh = (jax.lax.broadcasted_iota(jnp.int32, (s, _A), 1)
              == a_mod[:, None]).astype(f32)
    w_pick = jnp.dot(mod_oh, wn1a_ref[...], preferred_element_type=f32)
    node_oh = (jax.lax.broadcasted_iota(jnp.int32, (s, n), 1)
               == a_div[:, None]).astype(f32)
    act_add = (node_oh[:, :, None] * w_pick[:, None, :]).reshape(s * n, h)

    p = (jnp.dot(x, wn1x_ref[...], preferred_element_type=f32)
         + jnp.dot(agg, wn1g_ref[...], preferred_element_type=f32)
         + act_add + bn1_ref[...])
    hh = jnp.maximum(p, 0.0)
    h2 = jnp.dot(hh, wn2_ref[...], preferred_element_type=f32) + bn2_ref[...]
    hn = ln_relu(h2, gn_ref[...], bbn_ref[...])
    out = jnp.dot(hn, wn3_ref[...], preferred_element_type=f32) + bn3_ref[...]
    out_ref[...] = out.reshape(s, n, out.shape[-1])


def kernel(states, action, We1, be1, We2, be2, ge, bbe, We3, be3,
           Wn1, bn1, Wn2, bn2, gn, bbn, Wn3, bn3):
    b, n, d = states.shape
    h = We2.shape[0]
    s = 32
    nb = b // s

    ws, wt = We1[:d], We1[d:]
    wn1x = Wn1[:d]
    wn1a = Wn1[d:d + _A]
    wn1g = Wn1[d + _A:]
    act3 = action.astype(jnp.int32).reshape(nb, 1, s)

    row = lambda z: z.reshape(1, -1)
    full = lambda shp: pl.BlockSpec(shp, lambda i: (0,) * len(shp))

    out = pl.pallas_call(
        functools.partial(_gnn_block, s=s, n=n, d=d, h=h),
        grid=(nb,),
        in_specs=[
            pl.BlockSpec((s, n, d), lambda i: (i, 0, 0)),
            pl.BlockSpec((1, 1, s), lambda i: (i, 0, 0)),
            full((d, h)), full((d, h)), full((1, h)),
            full((h, h)), full((1, h)),
            full((1, h)), full((1, h)),
            full((h, h)), full((1, h)),
            full((d, h)), full((_A, h)), full((h, h)),
            full((1, h)),
            full((h, h)), full((1, h)),
            full((1, h)), full((1, h)),
            full((h, d)), full((1, d)),
        ],
        out_specs=pl.BlockSpec((s, n, d), lambda i: (i, 0, 0)),
        out_shape=jax.ShapeDtypeStruct((b, n, d), jnp.float32),
    )(states, act3, ws, wt, row(be1), We2, row(be2), row(ge), row(bbe),
      We3, row(be3), wn1x, wn1a, wn1g, row(bn1), Wn2, row(bn2), row(gn),
      row(bbn), Wn3, row(bn3))
    return out
